# baseline (speedup 1.0000x reference)
import functools

import jax
import jax.numpy as jnp
from jax import lax
from jax.experimental import pallas as pl
from jax.experimental.pallas import tpu as pltpu
from jax.experimental.pallas import tpu_sc as plsc

_V = 8192
_D = 1024
_B = 4 * 8192
_NW = 32
_NS = 16
_BPW = _B // _NW
_C = 16
_NCHUNK = _BPW // _C

_mesh = plsc.VectorSubcoreMesh(core_axis_name="c", subcore_axis_name="s")


@functools.partial(
    pl.kernel,
    mesh=_mesh,
    out_type=jax.ShapeDtypeStruct((_B, _D), jnp.float32),
    scratch_types=[
        pltpu.VMEM((_NCHUNK, _C), jnp.int32),
        pltpu.VMEM((_C, _D), jnp.float32),
        pltpu.VMEM_SHARED((_NS, _C, _D), jnp.float32),
        pltpu.SemaphoreType.DMA,
        pltpu.SemaphoreType.DMA,
        pltpu.SemaphoreType.DMA,
    ],
)
def _gather_rows(idx_hbm, table_hbm, out_hbm, idx_v, rows_v, spm,
                 sem_g, sem_x, sem_s):
    cid = lax.axis_index("c")
    sid = lax.axis_index("s")
    wid = sid * 2 + cid
    base = wid * _BPW
    pltpu.sync_copy(idx_hbm.at[wid], idx_v)

    def body(ci, carry):
        pltpu.async_copy(table_hbm.at[idx_v.at[ci]], rows_v, sem_g)
        pltpu.make_async_copy(
            table_hbm.at[idx_v.at[ci]], rows_v, sem_g).wait()
        pltpu.async_copy(rows_v, spm.at[sid], sem_x)
        pltpu.make_async_copy(rows_v, spm.at[sid], sem_x).wait()
        dst = out_hbm.at[pl.ds(base + ci * _C, _C)]
        pltpu.async_copy(spm.at[sid], dst, sem_s)
        pltpu.make_async_copy(spm.at[sid], dst, sem_s).wait()
        return carry

    lax.fori_loop(0, _NCHUNK, body, 0)


def kernel(position_ids, table):
    idx = position_ids.reshape(_NW, _NCHUNK, _C).astype(jnp.int32)
    out = _gather_rows(idx, table)
    return out.reshape(position_ids.shape + (_D,))
